# baseline (speedup 1.0000x reference)
import functools

import jax
import jax.numpy as jnp
from jax import lax
from jax.experimental import pallas as pl
from jax.experimental.pallas import tpu as pltpu
from jax.experimental.pallas import tpu_sc as plsc

N = 10000
EDG = 32000
DIN = 1024
F = 512
H = 8
NC = 2
NS = 16
EPT = 2048
SUB = 128
NSUB = EPT // SUB
NPAD = 10240
RPT = NPAD // NS
QW = 64
NQ = H
ZB = 64
NZB = RPT // ZB


def _proj_body(x_ref, w_ref, b_ref, a_ref, y_ref, al_ref):
    y = jnp.dot(x_ref[...].astype(jnp.bfloat16), w_ref[...].astype(jnp.bfloat16),
                preferred_element_type=jnp.float32)
    y = y + b_ref[...]
    y_ref[...] = y
    al_ref[...] = jnp.dot(y, a_ref[...], preferred_element_type=jnp.float32)


def _proj(x, w, b, a):
    B = 1000
    return pl.pallas_call(
        _proj_body,
        grid=(N // B,),
        in_specs=[
            pl.BlockSpec((B, DIN), lambda i: (i, 0)),
            pl.BlockSpec((DIN, F), lambda i: (0, 0)),
            pl.BlockSpec((1, F), lambda i: (0, 0)),
            pl.BlockSpec((F, 16), lambda i: (0, 0)),
        ],
        out_specs=[
            pl.BlockSpec((B, F), lambda i: (i, 0)),
            pl.BlockSpec((B, 16), lambda i: (i, 0)),
        ],
        out_shape=[
            jax.ShapeDtypeStruct((N, F), jnp.float32),
            jax.ShapeDtypeStruct((N, 16), jnp.float32),
        ],
    )(x, w, b, a)


def _fin_body(y_ref, den_ref, erep_ref, w_ref, b_ref, o_ref):
    r = 1.0 / (den_ref[...] + 1e-16)
    rep = jnp.dot(r, erep_ref[...], preferred_element_type=jnp.float32)
    y = jnp.maximum(y_ref[...] * rep, 0.0)
    o_ref[...] = (
        jnp.dot(y.astype(jnp.bfloat16), w_ref[...].astype(jnp.bfloat16),
                preferred_element_type=jnp.float32)
        + b_ref[...]
    )


def _fin(y, den, erep, w, b, toff):
    B = 80
    return pl.pallas_call(
        _fin_body,
        grid=(N // B,),
        in_specs=[
            pl.BlockSpec((B, F), lambda i: (toff + i, 0)),
            pl.BlockSpec((B, 16), lambda i: (toff + i, 0)),
            pl.BlockSpec((16, F), lambda i: (0, 0)),
            pl.BlockSpec((F, F), lambda i: (0, 0)),
            pl.BlockSpec((1, F), lambda i: (0, 0)),
        ],
        out_specs=pl.BlockSpec((B, F), lambda i: (i, 0)),
        out_shape=jax.ShapeDtypeStruct((N, F), jnp.float32),
    )(y, den, erep, w, b)


_mesh = plsc.VectorSubcoreMesh(
    core_axis_name="c", subcore_axis_name="s", num_cores=NC, num_subcores=NS
)


@functools.partial(
    pl.kernel,
    out_type=[
        jax.ShapeDtypeStruct((NC * NPAD, F), jnp.float32),
        jax.ShapeDtypeStruct((NC * NPAD, 16), jnp.float32),
    ],
    mesh=_mesh,
    compiler_params=pltpu.CompilerParams(use_tc_tiling_on_sc=False),
    scratch_types=[
        pltpu.VMEM_SHARED((NPAD, 16), jnp.float32),
        pltpu.VMEM_SHARED((NPAD, QW), jnp.float32),
        pltpu.VMEM((EPT, 16), jnp.float32),
        pltpu.VMEM((SUB, 16), jnp.float32),
        pltpu.VMEM((SUB, QW), jnp.float32),
        pltpu.VMEM((SUB, QW), jnp.float32),
        pltpu.VMEM((NSUB, SUB), jnp.int32),
        pltpu.VMEM((NSUB, SUB), jnp.int32),
        pltpu.VMEM((NSUB, SUB), jnp.int32),
        pltpu.VMEM((ZB, QW), jnp.float32),
        pltpu.VMEM((ZB, 16), jnp.float32),
        pltpu.SemaphoreType.DMA,
        pltpu.SemaphoreType.DMA,
        pltpu.SemaphoreType.DMA,
        pltpu.SemaphoreType.DMA,
    ],
)
def _sc_edge(atab, xtab, edg, out, dout,
             den_sp, slab_sp, wbuf, dch, xg0, xg1, sidx, didx, oidx,
             zb, zbd, gsem0, gsem1, ssem0, ssem1):
    c = lax.axis_index("c")
    s = lax.axis_index("s")
    rowbase = s * RPT
    zero16 = jnp.zeros((16,), jnp.float32)
    xgs = (xg0, xg1)
    gsems = (gsem0, gsem1)
    ssems = (ssem0, ssem1)

    def _offset_idx(base_ref, off):
        def f(j, _):
            for v in range(SUB // 16):
                oidx[j, pl.ds(v * 16, 16)] = base_ref[j, pl.ds(v * 16, 16)] + off
            return 0
        lax.fori_loop(0, NSUB, f, 0)

    def _zb_fill(i, _):
        zb[i // (QW // 16), pl.ds((i % (QW // 16)) * 16, 16)] = zero16
        return 0
    lax.fori_loop(0, ZB * (QW // 16), _zb_fill, 0)

    def _zbd_fill(i, _):
        zbd[i, ...] = zero16
        return 0
    lax.fori_loop(0, ZB, _zbd_fill, 0)

    pltpu.sync_copy(edg.at[pl.ds(((c * 2 + 0) * NS + s) * NSUB, NSUB)], sidx)
    pltpu.sync_copy(edg.at[pl.ds(((c * 2 + 1) * NS + s) * NSUB, NSUB)], didx)

    for k in range(NZB):
        pltpu.sync_copy(zbd, den_sp.at[pl.ds(rowbase + k * ZB, ZB)])
    plsc.subcore_barrier()

    _offset_idx(sidx, c * (2 * NPAD))

    def _alpha_src(j, _):
        pltpu.sync_copy(atab.at[oidx.at[j]], wbuf.at[pl.ds(j * SUB, SUB)])
        return 0
    lax.fori_loop(0, NSUB, _alpha_src, 0)

    _offset_idx(didx, c * (2 * NPAD) + NPAD)

    def _alpha(j, _):
        pltpu.sync_copy(atab.at[oidx.at[j]], dch)

        def _expw(e, _):
            a = wbuf[j * SUB + e, ...] + dch[e, ...]
            a = jnp.maximum(a, a * 0.2)
            wbuf[j * SUB + e, ...] = jnp.exp(a)
            return 0
        lax.fori_loop(0, SUB, _expw, 0)
        pltpu.sync_copy(wbuf.at[pl.ds(j * SUB, SUB)], den_sp.at[didx.at[j]],
                        add=True)
        return 0
    lax.fori_loop(0, NSUB, _alpha, 0)
    plsc.subcore_barrier()

    pltpu.sync_copy(den_sp.at[pl.ds(rowbase, RPT)],
                    dout.at[pl.ds(c * NPAD + rowbase, RPT)])

    for q in range(NQ):
        for k in range(NZB):
            pltpu.sync_copy(zb, slab_sp.at[pl.ds(rowbase + k * ZB, ZB)])
        _offset_idx(sidx, (c * H + q) * N)
        plsc.subcore_barrier()

        pltpu.async_copy(xtab.at[oidx.at[0]], xg0, gsem0)

        def _scale_buf(xgb, jbase):
            def _scale(e, _):
                ww = wbuf[jbase * SUB + e, ...][q]
                for v in range(QW // 16):
                    xgb[e, pl.ds(v * 16, 16)] = xgb[e, pl.ds(v * 16, 16)] * ww
                return 0
            lax.fori_loop(0, SUB, _scale, 0)

        def _msg2(j2, _):
            j = j2 * 2
            pltpu.make_async_copy(xtab.at[oidx.at[j]], xg0, gsem0).wait()

            @pl.when(j2 >= 1)
            def _():
                pltpu.make_async_copy(xg1, slab_sp.at[didx.at[j - 1]],
                                      ssem1).wait()
            pltpu.async_copy(xtab.at[oidx.at[j + 1]], xg1, gsem1)
            _scale_buf(xg0, j)
            pltpu.async_copy(xg0, slab_sp.at[didx.at[j]], ssem0, add=True)
            pltpu.make_async_copy(xtab.at[oidx.at[j + 1]], xg1, gsem1).wait()
            pltpu.make_async_copy(xg0, slab_sp.at[didx.at[j]], ssem0).wait()

            @pl.when(j2 + 1 < NSUB // 2)
            def _():
                pltpu.async_copy(xtab.at[oidx.at[j + 2]], xg0, gsem0)
            _scale_buf(xg1, j + 1)
            pltpu.async_copy(xg1, slab_sp.at[didx.at[j + 1]], ssem1, add=True)
            return 0
        lax.fori_loop(0, NSUB // 2, _msg2, 0)
        pltpu.make_async_copy(xg1, slab_sp.at[didx.at[NSUB - 1]],
                              ssem1).wait()
        plsc.subcore_barrier()

        pltpu.sync_copy(slab_sp.at[pl.ds(rowbase, RPT)],
                        out.at[pl.ds(c * NPAD + rowbase, RPT),
                               pl.ds(q * QW, QW)])


def _blockdiag(att):
    return (att[:, :, None] * jnp.eye(H, dtype=att.dtype)[:, None, :]).reshape(F, H)


@jax.jit
def kernel(x_promoter, x_enhancer, edge_index_pe, edge_index_ep,
           proj_p_W, proj_p_b, proj_e_W, proj_e_b,
           att_src_pe, att_dst_pe, att_src_ep, att_dst_ep,
           k_lin_W, k_lin_b, q, lin1_W, lin1_b):
    a_p = jnp.concatenate([_blockdiag(att_src_pe), _blockdiag(att_dst_ep)], axis=1)
    a_e = jnp.concatenate([_blockdiag(att_dst_pe), _blockdiag(att_src_ep)], axis=1)
    yp, alp = _proj(x_promoter, proj_p_W, proj_p_b.reshape(1, F), a_p)
    ye, ale = _proj(x_enhancer, proj_e_W, proj_e_b.reshape(1, F), a_e)
    z8 = jnp.zeros((N, 8), jnp.float32)
    pad = ((0, NPAD - N), (0, 0))
    atab = jnp.concatenate([
        jnp.pad(jnp.concatenate([alp[:, :8], z8], axis=1), pad),
        jnp.pad(jnp.concatenate([ale[:, :8], z8], axis=1), pad),
        jnp.pad(jnp.concatenate([ale[:, 8:], z8], axis=1), pad),
        jnp.pad(jnp.concatenate([alp[:, 8:], z8], axis=1), pad),
    ], axis=0)
    def _split(e, fill):
        return jnp.pad(e.reshape(NS, EDG // NS), ((0, 0), (0, EPT - EDG // NS)),
                       constant_values=fill)
    edg = jnp.stack([
        jnp.stack([_split(edge_index_pe[0], 0), _split(edge_index_pe[1], N)]),
        jnp.stack([_split(edge_index_ep[0], 0), _split(edge_index_ep[1], N)]),
    ]).reshape(NC * 2 * NS * NSUB, SUB)
    xtab = jnp.concatenate([
        yp.reshape(N, H, QW).transpose(1, 0, 2).reshape(H * N, QW),
        ye.reshape(N, H, QW).transpose(1, 0, 2).reshape(H * N, QW),
    ], axis=0)
    han, den = _sc_edge(atab, xtab, edg)
    erep = jnp.repeat(jnp.eye(16, dtype=jnp.float32), 64, axis=1)[:, :F]
    b1 = lin1_b.reshape(1, F)
    out_e = _fin(han, den, erep, lin1_W, b1, 0)
    out_p = _fin(han, den, erep, lin1_W, b1, NPAD // 80)
    return out_p, out_e


# speedup vs baseline: 1.1276x; 1.1276x over previous
import functools

import jax
import jax.numpy as jnp
from jax import lax
from jax.experimental import pallas as pl
from jax.experimental.pallas import tpu as pltpu
from jax.experimental.pallas import tpu_sc as plsc

N = 10000
EDG = 32000
DIN = 1024
F = 512
H = 8
NC = 2
NS = 16
EPT = 2048
SUB = 128
NSUB = EPT // SUB
NPAD = 10240
RPT = NPAD // NS
QW = 64
NQ = H
ZB = 64
NZB = RPT // ZB


def _proj_body(x_ref, w_ref, b_ref, a_ref, y_ref, al_ref):
    y = jnp.dot(x_ref[...].astype(jnp.bfloat16), w_ref[...].astype(jnp.bfloat16),
                preferred_element_type=jnp.float32)
    y = y + b_ref[...]
    y_ref[...] = y
    al_ref[...] = jnp.dot(y, a_ref[...], preferred_element_type=jnp.float32)


def _proj(x, w, b, a):
    B = 1000
    return pl.pallas_call(
        _proj_body,
        grid=(N // B,),
        in_specs=[
            pl.BlockSpec((B, DIN), lambda i: (i, 0)),
            pl.BlockSpec((DIN, F), lambda i: (0, 0)),
            pl.BlockSpec((1, F), lambda i: (0, 0)),
            pl.BlockSpec((F, 16), lambda i: (0, 0)),
        ],
        out_specs=[
            pl.BlockSpec((B, F), lambda i: (i, 0)),
            pl.BlockSpec((B, 16), lambda i: (i, 0)),
        ],
        out_shape=[
            jax.ShapeDtypeStruct((N, F), jnp.float32),
            jax.ShapeDtypeStruct((N, 16), jnp.float32),
        ],
    )(x, w, b, a)


def _fin_body(y_ref, den_ref, erep_ref, w_ref, b_ref, o_ref):
    r = 1.0 / (den_ref[...] + 1e-16)
    rep = jnp.dot(r, erep_ref[...], preferred_element_type=jnp.float32)
    y = jnp.maximum(y_ref[...] * rep, 0.0)
    o_ref[...] = (
        jnp.dot(y.astype(jnp.bfloat16), w_ref[...].astype(jnp.bfloat16),
                preferred_element_type=jnp.float32)
        + b_ref[...]
    )


def _fin(y, den, erep, w, b):
    B = 1024
    M = y.shape[0]
    return pl.pallas_call(
        _fin_body,
        grid=(M // B,),
        in_specs=[
            pl.BlockSpec((B, F), lambda i: (i, 0)),
            pl.BlockSpec((B, 16), lambda i: (i, 0)),
            pl.BlockSpec((16, F), lambda i: (0, 0)),
            pl.BlockSpec((F, F), lambda i: (0, 0)),
            pl.BlockSpec((1, F), lambda i: (0, 0)),
        ],
        out_specs=pl.BlockSpec((B, F), lambda i: (i, 0)),
        out_shape=jax.ShapeDtypeStruct((M, F), jnp.float32),
    )(y, den, erep, w, b)


_mesh = plsc.VectorSubcoreMesh(
    core_axis_name="c", subcore_axis_name="s", num_cores=NC, num_subcores=NS
)


@functools.partial(
    pl.kernel,
    out_type=[
        jax.ShapeDtypeStruct((NC * NPAD, F), jnp.float32),
        jax.ShapeDtypeStruct((NC * NPAD, 16), jnp.float32),
    ],
    mesh=_mesh,
    compiler_params=pltpu.CompilerParams(use_tc_tiling_on_sc=False),
    scratch_types=[
        pltpu.VMEM_SHARED((NPAD, 16), jnp.float32),
        pltpu.VMEM_SHARED((NPAD, QW), jnp.float32),
        pltpu.VMEM((EPT, 16), jnp.float32),
        pltpu.VMEM((SUB, 16), jnp.float32),
        pltpu.VMEM((SUB, QW), jnp.float32),
        pltpu.VMEM((SUB, QW), jnp.float32),
        pltpu.VMEM((NSUB, SUB), jnp.int32),
        pltpu.VMEM((NSUB, SUB), jnp.int32),
        pltpu.VMEM((NSUB, SUB), jnp.int32),
        pltpu.VMEM((ZB, QW), jnp.float32),
        pltpu.VMEM((ZB, 16), jnp.float32),
        pltpu.SemaphoreType.DMA,
        pltpu.SemaphoreType.DMA,
        pltpu.SemaphoreType.DMA,
        pltpu.SemaphoreType.DMA,
    ],
)
def _sc_edge(atab, xtab, edg, out, dout,
             den_sp, slab_sp, wbuf, dch, xg0, xg1, sidx, didx, oidx,
             zb, zbd, gsem0, gsem1, ssem0, ssem1):
    c = lax.axis_index("c")
    s = lax.axis_index("s")
    rowbase = s * RPT
    zero16 = jnp.zeros((16,), jnp.float32)
    xgs = (xg0, xg1)
    gsems = (gsem0, gsem1)
    ssems = (ssem0, ssem1)

    def _offset_idx(base_ref, off):
        def f(j, _):
            for v in range(SUB // 16):
                oidx[j, pl.ds(v * 16, 16)] = base_ref[j, pl.ds(v * 16, 16)] + off
            return 0
        lax.fori_loop(0, NSUB, f, 0)

    def _zb_fill(i, _):
        zb[i // (QW // 16), pl.ds((i % (QW // 16)) * 16, 16)] = zero16
        return 0
    lax.fori_loop(0, ZB * (QW // 16), _zb_fill, 0)

    def _zbd_fill(i, _):
        zbd[i, ...] = zero16
        return 0
    lax.fori_loop(0, ZB, _zbd_fill, 0)

    pltpu.sync_copy(edg.at[pl.ds(((c * 2 + 0) * NS + s) * NSUB, NSUB)], sidx)
    pltpu.sync_copy(edg.at[pl.ds(((c * 2 + 1) * NS + s) * NSUB, NSUB)], didx)

    for k in range(NZB):
        pltpu.sync_copy(zbd, den_sp.at[pl.ds(rowbase + k * ZB, ZB)])
    plsc.subcore_barrier()

    _offset_idx(sidx, c * (2 * NPAD))

    def _alpha_src(j, _):
        pltpu.sync_copy(atab.at[oidx.at[j]], wbuf.at[pl.ds(j * SUB, SUB)])
        return 0
    lax.fori_loop(0, NSUB, _alpha_src, 0)

    _offset_idx(didx, c * (2 * NPAD) + NPAD)

    def _alpha(j, _):
        pltpu.sync_copy(atab.at[oidx.at[j]], dch)

        def _expw(e, _):
            a = wbuf[j * SUB + e, ...] + dch[e, ...]
            a = jnp.maximum(a, a * 0.2)
            wbuf[j * SUB + e, ...] = jnp.exp(a)
            return 0
        lax.fori_loop(0, SUB, _expw, 0)
        pltpu.sync_copy(wbuf.at[pl.ds(j * SUB, SUB)], den_sp.at[didx.at[j]],
                        add=True)
        return 0
    lax.fori_loop(0, NSUB, _alpha, 0)
    plsc.subcore_barrier()

    pltpu.sync_copy(den_sp.at[pl.ds(rowbase, RPT)],
                    dout.at[pl.ds(c * NPAD + rowbase, RPT)])

    for q in range(NQ):
        for k in range(NZB):
            pltpu.sync_copy(zb, slab_sp.at[pl.ds(rowbase + k * ZB, ZB)])
        _offset_idx(sidx, (c * H + q) * N)
        plsc.subcore_barrier()

        pltpu.async_copy(xtab.at[oidx.at[0]], xg0, gsem0)

        def _scale_buf(xgb, jbase):
            def _scale(e, _):
                ww = wbuf[jbase * SUB + e, ...][q]
                for v in range(QW // 16):
                    xgb[e, pl.ds(v * 16, 16)] = xgb[e, pl.ds(v * 16, 16)] * ww
                return 0
            lax.fori_loop(0, SUB, _scale, 0)

        def _msg2(j2, _):
            j = j2 * 2
            pltpu.make_async_copy(xtab.at[oidx.at[j]], xg0, gsem0).wait()

            @pl.when(j2 >= 1)
            def _():
                pltpu.make_async_copy(xg1, slab_sp.at[didx.at[j - 1]],
                                      ssem1).wait()
            pltpu.async_copy(xtab.at[oidx.at[j + 1]], xg1, gsem1)
            _scale_buf(xg0, j)
            pltpu.async_copy(xg0, slab_sp.at[didx.at[j]], ssem0, add=True)
            pltpu.make_async_copy(xtab.at[oidx.at[j + 1]], xg1, gsem1).wait()
            pltpu.make_async_copy(xg0, slab_sp.at[didx.at[j]], ssem0).wait()

            @pl.when(j2 + 1 < NSUB // 2)
            def _():
                pltpu.async_copy(xtab.at[oidx.at[j + 2]], xg0, gsem0)
            _scale_buf(xg1, j + 1)
            pltpu.async_copy(xg1, slab_sp.at[didx.at[j + 1]], ssem1, add=True)
            return 0
        lax.fori_loop(0, NSUB // 2, _msg2, 0)
        pltpu.make_async_copy(xg1, slab_sp.at[didx.at[NSUB - 1]],
                              ssem1).wait()
        plsc.subcore_barrier()

        pltpu.sync_copy(slab_sp.at[pl.ds(rowbase, RPT)],
                        out.at[pl.ds(c * NPAD + rowbase, RPT),
                               pl.ds(q * QW, QW)])


def _blockdiag(att):
    return (att[:, :, None] * jnp.eye(H, dtype=att.dtype)[:, None, :]).reshape(F, H)


@jax.jit
def kernel(x_promoter, x_enhancer, edge_index_pe, edge_index_ep,
           proj_p_W, proj_p_b, proj_e_W, proj_e_b,
           att_src_pe, att_dst_pe, att_src_ep, att_dst_ep,
           k_lin_W, k_lin_b, q, lin1_W, lin1_b):
    a_p = jnp.concatenate([_blockdiag(att_src_pe), _blockdiag(att_dst_ep)], axis=1)
    a_e = jnp.concatenate([_blockdiag(att_dst_pe), _blockdiag(att_src_ep)], axis=1)
    yp, alp = _proj(x_promoter, proj_p_W, proj_p_b.reshape(1, F), a_p)
    ye, ale = _proj(x_enhancer, proj_e_W, proj_e_b.reshape(1, F), a_e)
    z8 = jnp.zeros((N, 8), jnp.float32)
    pad = ((0, NPAD - N), (0, 0))
    atab = jnp.concatenate([
        jnp.pad(jnp.concatenate([alp[:, :8], z8], axis=1), pad),
        jnp.pad(jnp.concatenate([ale[:, :8], z8], axis=1), pad),
        jnp.pad(jnp.concatenate([ale[:, 8:], z8], axis=1), pad),
        jnp.pad(jnp.concatenate([alp[:, 8:], z8], axis=1), pad),
    ], axis=0)
    def _split(e, fill):
        return jnp.pad(e.reshape(NS, EDG // NS), ((0, 0), (0, EPT - EDG // NS)),
                       constant_values=fill)
    edg = jnp.stack([
        jnp.stack([_split(edge_index_pe[0], 0), _split(edge_index_pe[1], N)]),
        jnp.stack([_split(edge_index_ep[0], 0), _split(edge_index_ep[1], N)]),
    ]).reshape(NC * 2 * NS * NSUB, SUB)
    xtab = jnp.concatenate([
        yp.reshape(N, H, QW).transpose(1, 0, 2).reshape(H * N, QW),
        ye.reshape(N, H, QW).transpose(1, 0, 2).reshape(H * N, QW),
    ], axis=0)
    han, den = _sc_edge(atab, xtab, edg)
    erep = jnp.repeat(jnp.eye(16, dtype=jnp.float32), 64, axis=1)[:, :F]
    fin = _fin(han, den, erep, lin1_W, lin1_b.reshape(1, F))
    out_e = fin[:N]
    out_p = fin[NPAD:NPAD + N]
    return out_p, out_e


# speedup vs baseline: 1.1596x; 1.0284x over previous
import functools

import jax
import jax.numpy as jnp
from jax import lax
from jax.experimental import pallas as pl
from jax.experimental.pallas import tpu as pltpu
from jax.experimental.pallas import tpu_sc as plsc

N = 10000
EDG = 32000
DIN = 1024
F = 512
H = 8
NC = 2
NS = 16
EPT = 2048
SUB = 128
NSUB = EPT // SUB
NPAD = 10240
RPT = NPAD // NS
QW = 64
NQ = H
ZB = 64
NZB = RPT // ZB


def _proj_body(x_ref, w_ref, b_ref, a_ref, y_ref, al_ref):
    y = jnp.dot(x_ref[...].astype(jnp.bfloat16), w_ref[...].astype(jnp.bfloat16),
                preferred_element_type=jnp.float32)
    y = y + b_ref[...]
    y_ref[...] = y
    al_ref[...] = jnp.dot(y, a_ref[...], preferred_element_type=jnp.float32)


def _proj(x, w, b, a):
    B = 2000
    return pl.pallas_call(
        _proj_body,
        grid=(N // B,),
        in_specs=[
            pl.BlockSpec((B, DIN), lambda i: (i, 0)),
            pl.BlockSpec((DIN, F), lambda i: (0, 0)),
            pl.BlockSpec((1, F), lambda i: (0, 0)),
            pl.BlockSpec((F, 16), lambda i: (0, 0)),
        ],
        out_specs=[
            pl.BlockSpec((B, F), lambda i: (i, 0)),
            pl.BlockSpec((B, 16), lambda i: (i, 0)),
        ],
        out_shape=[
            jax.ShapeDtypeStruct((N, F), jnp.float32),
            jax.ShapeDtypeStruct((N, 16), jnp.float32),
        ],
    )(x, w, b, a)


def _fin_body(y_ref, den_ref, erep_ref, w_ref, b_ref, o_ref):
    r = 1.0 / (den_ref[...] + 1e-16)
    rep = jnp.dot(r, erep_ref[...], preferred_element_type=jnp.float32)
    y = jnp.maximum(y_ref[...] * rep, 0.0)
    o_ref[...] = (
        jnp.dot(y.astype(jnp.bfloat16), w_ref[...].astype(jnp.bfloat16),
                preferred_element_type=jnp.float32)
        + b_ref[...]
    )


def _fin(y, den, erep, w, b):
    B = 1024
    M = y.shape[0]
    return pl.pallas_call(
        _fin_body,
        grid=(M // B,),
        in_specs=[
            pl.BlockSpec((B, F), lambda i: (i, 0)),
            pl.BlockSpec((B, 16), lambda i: (i, 0)),
            pl.BlockSpec((16, F), lambda i: (0, 0)),
            pl.BlockSpec((F, F), lambda i: (0, 0)),
            pl.BlockSpec((1, F), lambda i: (0, 0)),
        ],
        out_specs=pl.BlockSpec((B, F), lambda i: (i, 0)),
        out_shape=jax.ShapeDtypeStruct((M, F), jnp.float32),
    )(y, den, erep, w, b)


_mesh = plsc.VectorSubcoreMesh(
    core_axis_name="c", subcore_axis_name="s", num_cores=NC, num_subcores=NS
)


@functools.partial(
    pl.kernel,
    out_type=[
        jax.ShapeDtypeStruct((NC * NPAD, F), jnp.float32),
        jax.ShapeDtypeStruct((NC * NPAD, 16), jnp.float32),
    ],
    mesh=_mesh,
    compiler_params=pltpu.CompilerParams(use_tc_tiling_on_sc=False),
    scratch_types=[
        pltpu.VMEM_SHARED((NPAD, 16), jnp.float32),
        pltpu.VMEM_SHARED((NPAD, QW), jnp.float32),
        pltpu.VMEM((EPT, 16), jnp.float32),
        pltpu.VMEM((SUB, 16), jnp.float32),
        pltpu.VMEM((SUB, 16), jnp.float32),
        pltpu.VMEM((SUB, QW), jnp.float32),
        pltpu.VMEM((SUB, QW), jnp.float32),
        pltpu.VMEM((NSUB, SUB), jnp.int32),
        pltpu.VMEM((NSUB, SUB), jnp.int32),
        pltpu.VMEM((NSUB, SUB), jnp.int32),
        pltpu.VMEM((ZB, QW), jnp.float32),
        pltpu.VMEM((ZB, 16), jnp.float32),
        pltpu.SemaphoreType.DMA,
        pltpu.SemaphoreType.DMA,
        pltpu.SemaphoreType.DMA,
        pltpu.SemaphoreType.DMA,
        pltpu.SemaphoreType.DMA,
        pltpu.SemaphoreType.DMA,
    ],
)
def _sc_edge(atab, xtab, edg, out, dout,
             den_sp, slab_sp, wbuf, dch0, dch1, xg0, xg1, sidx, didx, oidx,
             zb, zbd, gsem0, gsem1, ssem0, ssem1, asem0, asem1):
    c = lax.axis_index("c")
    s = lax.axis_index("s")
    rowbase = s * RPT
    zero16 = jnp.zeros((16,), jnp.float32)
    xgs = (xg0, xg1)
    gsems = (gsem0, gsem1)
    ssems = (ssem0, ssem1)

    def _offset_idx(base_ref, off):
        def f(j, _):
            for v in range(SUB // 16):
                oidx[j, pl.ds(v * 16, 16)] = base_ref[j, pl.ds(v * 16, 16)] + off
            return 0
        lax.fori_loop(0, NSUB, f, 0)

    def _zb_fill(i, _):
        zb[i // (QW // 16), pl.ds((i % (QW // 16)) * 16, 16)] = zero16
        return 0
    lax.fori_loop(0, ZB * (QW // 16), _zb_fill, 0)

    def _zbd_fill(i, _):
        zbd[i, ...] = zero16
        return 0
    lax.fori_loop(0, ZB, _zbd_fill, 0)

    pltpu.sync_copy(edg.at[pl.ds(((c * 2 + 0) * NS + s) * NSUB, NSUB)], sidx)
    pltpu.sync_copy(edg.at[pl.ds(((c * 2 + 1) * NS + s) * NSUB, NSUB)], didx)

    for k in range(NZB):
        pltpu.sync_copy(zbd, den_sp.at[pl.ds(rowbase + k * ZB, ZB)])
    plsc.subcore_barrier()

    _offset_idx(sidx, c * (2 * NPAD))

    def _alpha_src(j, _):
        pltpu.async_copy(atab.at[oidx.at[j]], wbuf.at[pl.ds(j * SUB, SUB)],
                         gsem0)
        return 0
    lax.fori_loop(0, NSUB, _alpha_src, 0)

    def _alpha_src_drain(j, _):
        pltpu.make_async_copy(atab.at[oidx.at[j]],
                              wbuf.at[pl.ds(j * SUB, SUB)], gsem0).wait()
        return 0
    lax.fori_loop(0, NSUB, _alpha_src_drain, 0)

    _offset_idx(didx, c * (2 * NPAD) + NPAD)

    def _expw_chunk(j, dch):
        def _expw(e, _):
            a = wbuf[j * SUB + e, ...] + dch[e, ...]
            a = jnp.maximum(a, a * 0.2)
            wbuf[j * SUB + e, ...] = jnp.exp(a)
            return 0
        lax.fori_loop(0, SUB, _expw, 0)
        pltpu.async_copy(wbuf.at[pl.ds(j * SUB, SUB)], den_sp.at[didx.at[j]],
                        ssem0, add=True)

    pltpu.async_copy(atab.at[oidx.at[0]], dch0, asem0)

    def _alpha2(j2, _):
        j = j2 * 2
        pltpu.make_async_copy(atab.at[oidx.at[j]], dch0, asem0).wait()
        pltpu.async_copy(atab.at[oidx.at[j + 1]], dch1, asem1)
        _expw_chunk(j, dch0)
        pltpu.make_async_copy(atab.at[oidx.at[j + 1]], dch1, asem1).wait()

        @pl.when(j2 + 1 < NSUB // 2)
        def _():
            pltpu.async_copy(atab.at[oidx.at[j + 2]], dch0, asem0)
        _expw_chunk(j + 1, dch1)
        return 0
    lax.fori_loop(0, NSUB // 2, _alpha2, 0)

    def _dscat_drain(j, _):
        pltpu.make_async_copy(wbuf.at[pl.ds(j * SUB, SUB)],
                              den_sp.at[didx.at[j]], ssem0).wait()
        return 0
    lax.fori_loop(0, NSUB, _dscat_drain, 0)
    plsc.subcore_barrier()

    pltpu.sync_copy(den_sp.at[pl.ds(rowbase, RPT)],
                    dout.at[pl.ds(c * NPAD + rowbase, RPT)])

    for q in range(NQ):
        for k in range(NZB):
            pltpu.sync_copy(zb, slab_sp.at[pl.ds(rowbase + k * ZB, ZB)])
        _offset_idx(sidx, (c * H + q) * N)
        plsc.subcore_barrier()

        pltpu.async_copy(xtab.at[oidx.at[0]], xg0, gsem0)

        def _scale_buf(xgb, jbase):
            def _scale(e, _):
                ww = wbuf[jbase * SUB + e, ...][q]
                for v in range(QW // 16):
                    xgb[e, pl.ds(v * 16, 16)] = xgb[e, pl.ds(v * 16, 16)] * ww
                return 0
            lax.fori_loop(0, SUB, _scale, 0)

        def _msg2(j2, _):
            j = j2 * 2
            pltpu.make_async_copy(xtab.at[oidx.at[j]], xg0, gsem0).wait()

            @pl.when(j2 >= 1)
            def _():
                pltpu.make_async_copy(xg1, slab_sp.at[didx.at[j - 1]],
                                      ssem1).wait()
            pltpu.async_copy(xtab.at[oidx.at[j + 1]], xg1, gsem1)
            _scale_buf(xg0, j)
            pltpu.async_copy(xg0, slab_sp.at[didx.at[j]], ssem0, add=True)
            pltpu.make_async_copy(xtab.at[oidx.at[j + 1]], xg1, gsem1).wait()
            pltpu.make_async_copy(xg0, slab_sp.at[didx.at[j]], ssem0).wait()

            @pl.when(j2 + 1 < NSUB // 2)
            def _():
                pltpu.async_copy(xtab.at[oidx.at[j + 2]], xg0, gsem0)
            _scale_buf(xg1, j + 1)
            pltpu.async_copy(xg1, slab_sp.at[didx.at[j + 1]], ssem1, add=True)
            return 0
        lax.fori_loop(0, NSUB // 2, _msg2, 0)
        pltpu.make_async_copy(xg1, slab_sp.at[didx.at[NSUB - 1]],
                              ssem1).wait()
        plsc.subcore_barrier()

        pltpu.sync_copy(slab_sp.at[pl.ds(rowbase, RPT)],
                        out.at[pl.ds(c * NPAD + rowbase, RPT),
                               pl.ds(q * QW, QW)])


def _blockdiag(att):
    return (att[:, :, None] * jnp.eye(H, dtype=att.dtype)[:, None, :]).reshape(F, H)


@jax.jit
def kernel(x_promoter, x_enhancer, edge_index_pe, edge_index_ep,
           proj_p_W, proj_p_b, proj_e_W, proj_e_b,
           att_src_pe, att_dst_pe, att_src_ep, att_dst_ep,
           k_lin_W, k_lin_b, q, lin1_W, lin1_b):
    a_p = jnp.concatenate([_blockdiag(att_src_pe), _blockdiag(att_dst_ep)], axis=1)
    a_e = jnp.concatenate([_blockdiag(att_dst_pe), _blockdiag(att_src_ep)], axis=1)
    yp, alp = _proj(x_promoter, proj_p_W, proj_p_b.reshape(1, F), a_p)
    ye, ale = _proj(x_enhancer, proj_e_W, proj_e_b.reshape(1, F), a_e)
    z8 = jnp.zeros((N, 8), jnp.float32)
    pad = ((0, NPAD - N), (0, 0))
    atab = jnp.concatenate([
        jnp.pad(jnp.concatenate([alp[:, :8], z8], axis=1), pad),
        jnp.pad(jnp.concatenate([ale[:, :8], z8], axis=1), pad),
        jnp.pad(jnp.concatenate([ale[:, 8:], z8], axis=1), pad),
        jnp.pad(jnp.concatenate([alp[:, 8:], z8], axis=1), pad),
    ], axis=0)
    def _split(e, fill):
        return jnp.pad(e.reshape(NS, EDG // NS), ((0, 0), (0, EPT - EDG // NS)),
                       constant_values=fill)
    edg = jnp.stack([
        jnp.stack([_split(edge_index_pe[0], 0), _split(edge_index_pe[1], N)]),
        jnp.stack([_split(edge_index_ep[0], 0), _split(edge_index_ep[1], N)]),
    ]).reshape(NC * 2 * NS * NSUB, SUB)
    xtab = jnp.concatenate([
        yp.reshape(N, H, QW).transpose(1, 0, 2).reshape(H * N, QW),
        ye.reshape(N, H, QW).transpose(1, 0, 2).reshape(H * N, QW),
    ], axis=0)
    han, den = _sc_edge(atab, xtab, edg)
    erep = jnp.repeat(jnp.eye(16, dtype=jnp.float32), 64, axis=1)[:, :F]
    fin = _fin(han, den, erep, lin1_W, lin1_b.reshape(1, F))
    out_e = fin[:N]
    out_p = fin[NPAD:NPAD + N]
    return out_p, out_e


# speedup vs baseline: 1.1681x; 1.0074x over previous
import functools

import jax
import jax.numpy as jnp
from jax import lax
from jax.experimental import pallas as pl
from jax.experimental.pallas import tpu as pltpu
from jax.experimental.pallas import tpu_sc as plsc

N = 10000
EDG = 32000
DIN = 1024
F = 512
H = 8
NC = 2
NS = 16
EPT = 2048
SUB = 128
NSUB = EPT // SUB
NPAD = 10240
RPT = NPAD // NS
QW = 64
NQ = H
ZB = 128
NZB = RPT // ZB


def _proj_body(x_ref, w_ref, b_ref, a_ref, y_ref, al_ref):
    y = jnp.dot(x_ref[...].astype(jnp.bfloat16), w_ref[...].astype(jnp.bfloat16),
                preferred_element_type=jnp.float32)
    y = y + b_ref[...]
    y_ref[...] = y
    al_ref[...] = jnp.dot(y, a_ref[...], preferred_element_type=jnp.float32)


def _proj(x, w, b, a):
    B = 2000
    return pl.pallas_call(
        _proj_body,
        grid=(N // B,),
        in_specs=[
            pl.BlockSpec((B, DIN), lambda i: (i, 0)),
            pl.BlockSpec((DIN, F), lambda i: (0, 0)),
            pl.BlockSpec((1, F), lambda i: (0, 0)),
            pl.BlockSpec((F, 16), lambda i: (0, 0)),
        ],
        out_specs=[
            pl.BlockSpec((B, F), lambda i: (i, 0)),
            pl.BlockSpec((B, 16), lambda i: (i, 0)),
        ],
        out_shape=[
            jax.ShapeDtypeStruct((N, F), jnp.float32),
            jax.ShapeDtypeStruct((N, 16), jnp.float32),
        ],
    )(x, w, b, a)


def _fin_body(y_ref, den_ref, erep_ref, w_ref, b_ref, o_ref):
    r = 1.0 / (den_ref[...] + 1e-16)
    rep = jnp.dot(r, erep_ref[...], preferred_element_type=jnp.float32)
    y = jnp.maximum(y_ref[...] * rep, 0.0)
    o_ref[...] = (
        jnp.dot(y.astype(jnp.bfloat16), w_ref[...].astype(jnp.bfloat16),
                preferred_element_type=jnp.float32)
        + b_ref[...]
    )


def _fin(y, den, erep, w, b):
    B = 1024
    M = y.shape[0]
    return pl.pallas_call(
        _fin_body,
        grid=(M // B,),
        in_specs=[
            pl.BlockSpec((B, F), lambda i: (i, 0)),
            pl.BlockSpec((B, 16), lambda i: (i, 0)),
            pl.BlockSpec((16, F), lambda i: (0, 0)),
            pl.BlockSpec((F, F), lambda i: (0, 0)),
            pl.BlockSpec((1, F), lambda i: (0, 0)),
        ],
        out_specs=pl.BlockSpec((B, F), lambda i: (i, 0)),
        out_shape=jax.ShapeDtypeStruct((M, F), jnp.float32),
    )(y, den, erep, w, b)


_mesh = plsc.VectorSubcoreMesh(
    core_axis_name="c", subcore_axis_name="s", num_cores=NC, num_subcores=NS
)


@functools.partial(
    pl.kernel,
    out_type=[
        jax.ShapeDtypeStruct((NC * NPAD, F), jnp.float32),
        jax.ShapeDtypeStruct((NC * NPAD, 16), jnp.float32),
    ],
    mesh=_mesh,
    compiler_params=pltpu.CompilerParams(use_tc_tiling_on_sc=False),
    scratch_types=[
        pltpu.VMEM_SHARED((NPAD, 16), jnp.float32),
        pltpu.VMEM_SHARED((NPAD, QW), jnp.float32),
        pltpu.VMEM((EPT, 16), jnp.float32),
        pltpu.VMEM((SUB, 16), jnp.float32),
        pltpu.VMEM((SUB, 16), jnp.float32),
        pltpu.VMEM((SUB, QW), jnp.float32),
        pltpu.VMEM((SUB, QW), jnp.float32),
        pltpu.VMEM((NSUB, SUB), jnp.int32),
        pltpu.VMEM((NSUB, SUB), jnp.int32),
        pltpu.VMEM((NSUB, SUB), jnp.int32),
        pltpu.VMEM((ZB, QW), jnp.float32),
        pltpu.VMEM((ZB, 16), jnp.float32),
        pltpu.SemaphoreType.DMA,
        pltpu.SemaphoreType.DMA,
        pltpu.SemaphoreType.DMA,
        pltpu.SemaphoreType.DMA,
        pltpu.SemaphoreType.DMA,
        pltpu.SemaphoreType.DMA,
    ],
)
def _sc_edge(atab, xtab, edg, out, dout,
             den_sp, slab_sp, wbuf, dch0, dch1, xg0, xg1, sidx, didx, oidx,
             zb, zbd, gsem0, gsem1, ssem0, ssem1, asem0, asem1):
    c = lax.axis_index("c")
    s = lax.axis_index("s")
    rowbase = s * RPT
    zero16 = jnp.zeros((16,), jnp.float32)
    xgs = (xg0, xg1)
    gsems = (gsem0, gsem1)
    ssems = (ssem0, ssem1)

    def _offset_idx(base_ref, off):
        def f(j, _):
            for v in range(SUB // 16):
                oidx[j, pl.ds(v * 16, 16)] = base_ref[j, pl.ds(v * 16, 16)] + off
            return 0
        lax.fori_loop(0, NSUB, f, 0)

    def _zb_fill(i, _):
        zb[i // (QW // 16), pl.ds((i % (QW // 16)) * 16, 16)] = zero16
        return 0
    lax.fori_loop(0, ZB * (QW // 16), _zb_fill, 0)

    def _zbd_fill(i, _):
        zbd[i, ...] = zero16
        return 0
    lax.fori_loop(0, ZB, _zbd_fill, 0)

    pltpu.sync_copy(edg.at[pl.ds(((c * 2 + 0) * NS + s) * NSUB, NSUB)], sidx)
    pltpu.sync_copy(edg.at[pl.ds(((c * 2 + 1) * NS + s) * NSUB, NSUB)], didx)

    for k in range(NZB):
        pltpu.sync_copy(zbd, den_sp.at[pl.ds(rowbase + k * ZB, ZB)])
    plsc.subcore_barrier()

    _offset_idx(sidx, c * (2 * NPAD))

    def _alpha_src(j, _):
        pltpu.async_copy(atab.at[oidx.at[j]], wbuf.at[pl.ds(j * SUB, SUB)],
                         gsem0)
        return 0
    lax.fori_loop(0, NSUB, _alpha_src, 0)

    def _alpha_src_drain(j, _):
        pltpu.make_async_copy(atab.at[oidx.at[j]],
                              wbuf.at[pl.ds(j * SUB, SUB)], gsem0).wait()
        return 0
    lax.fori_loop(0, NSUB, _alpha_src_drain, 0)

    _offset_idx(didx, c * (2 * NPAD) + NPAD)

    def _expw_chunk(j, dch):
        def _expw(e, _):
            a = wbuf[j * SUB + e, ...] + dch[e, ...]
            a = jnp.maximum(a, a * 0.2)
            wbuf[j * SUB + e, ...] = jnp.exp(a)
            return 0
        lax.fori_loop(0, SUB, _expw, 0)
        pltpu.async_copy(wbuf.at[pl.ds(j * SUB, SUB)], den_sp.at[didx.at[j]],
                        ssem0, add=True)

    pltpu.async_copy(atab.at[oidx.at[0]], dch0, asem0)

    def _alpha2(j2, _):
        j = j2 * 2
        pltpu.make_async_copy(atab.at[oidx.at[j]], dch0, asem0).wait()
        pltpu.async_copy(atab.at[oidx.at[j + 1]], dch1, asem1)
        _expw_chunk(j, dch0)
        pltpu.make_async_copy(atab.at[oidx.at[j + 1]], dch1, asem1).wait()

        @pl.when(j2 + 1 < NSUB // 2)
        def _():
            pltpu.async_copy(atab.at[oidx.at[j + 2]], dch0, asem0)
        _expw_chunk(j + 1, dch1)
        return 0
    lax.fori_loop(0, NSUB // 2, _alpha2, 0)

    def _dscat_drain(j, _):
        pltpu.make_async_copy(wbuf.at[pl.ds(j * SUB, SUB)],
                              den_sp.at[didx.at[j]], ssem0).wait()
        return 0
    lax.fori_loop(0, NSUB, _dscat_drain, 0)
    plsc.subcore_barrier()

    pltpu.sync_copy(den_sp.at[pl.ds(rowbase, RPT)],
                    dout.at[pl.ds(c * NPAD + rowbase, RPT)])

    for q in range(NQ):
        _offset_idx(sidx, (c * H + q) * N)
        pltpu.async_copy(xtab.at[oidx.at[0]], xg0, gsem0)
        for k in range(NZB):
            pltpu.sync_copy(zb, slab_sp.at[pl.ds(rowbase + k * ZB, ZB)])
        plsc.subcore_barrier()

        def _scale_buf(xgb, jbase):
            def _scale(e, _):
                ww = wbuf[jbase * SUB + e, ...][q]
                for v in range(QW // 16):
                    xgb[e, pl.ds(v * 16, 16)] = xgb[e, pl.ds(v * 16, 16)] * ww
                return 0
            lax.fori_loop(0, SUB, _scale, 0)

        def _msg2(j2, _):
            j = j2 * 2
            pltpu.make_async_copy(xtab.at[oidx.at[j]], xg0, gsem0).wait()

            @pl.when(j2 >= 1)
            def _():
                pltpu.make_async_copy(xg1, slab_sp.at[didx.at[j - 1]],
                                      ssem1).wait()
            pltpu.async_copy(xtab.at[oidx.at[j + 1]], xg1, gsem1)
            _scale_buf(xg0, j)
            pltpu.async_copy(xg0, slab_sp.at[didx.at[j]], ssem0, add=True)
            pltpu.make_async_copy(xtab.at[oidx.at[j + 1]], xg1, gsem1).wait()
            pltpu.make_async_copy(xg0, slab_sp.at[didx.at[j]], ssem0).wait()

            @pl.when(j2 + 1 < NSUB // 2)
            def _():
                pltpu.async_copy(xtab.at[oidx.at[j + 2]], xg0, gsem0)
            _scale_buf(xg1, j + 1)
            pltpu.async_copy(xg1, slab_sp.at[didx.at[j + 1]], ssem1, add=True)
            return 0
        lax.fori_loop(0, NSUB // 2, _msg2, 0)
        pltpu.make_async_copy(xg1, slab_sp.at[didx.at[NSUB - 1]],
                              ssem1).wait()
        plsc.subcore_barrier()

        pltpu.sync_copy(slab_sp.at[pl.ds(rowbase, RPT)],
                        out.at[pl.ds(c * NPAD + rowbase, RPT),
                               pl.ds(q * QW, QW)])


def _blockdiag(att):
    return (att[:, :, None] * jnp.eye(H, dtype=att.dtype)[:, None, :]).reshape(F, H)


@jax.jit
def kernel(x_promoter, x_enhancer, edge_index_pe, edge_index_ep,
           proj_p_W, proj_p_b, proj_e_W, proj_e_b,
           att_src_pe, att_dst_pe, att_src_ep, att_dst_ep,
           k_lin_W, k_lin_b, q, lin1_W, lin1_b):
    a_p = jnp.concatenate([_blockdiag(att_src_pe), _blockdiag(att_dst_ep)], axis=1)
    a_e = jnp.concatenate([_blockdiag(att_dst_pe), _blockdiag(att_src_ep)], axis=1)
    yp, alp = _proj(x_promoter, proj_p_W, proj_p_b.reshape(1, F), a_p)
    ye, ale = _proj(x_enhancer, proj_e_W, proj_e_b.reshape(1, F), a_e)
    z8 = jnp.zeros((N, 8), jnp.float32)
    pad = ((0, NPAD - N), (0, 0))
    atab = jnp.concatenate([
        jnp.pad(jnp.concatenate([alp[:, :8], z8], axis=1), pad),
        jnp.pad(jnp.concatenate([ale[:, :8], z8], axis=1), pad),
        jnp.pad(jnp.concatenate([ale[:, 8:], z8], axis=1), pad),
        jnp.pad(jnp.concatenate([alp[:, 8:], z8], axis=1), pad),
    ], axis=0)
    def _split(e, fill):
        return jnp.pad(e.reshape(NS, EDG // NS), ((0, 0), (0, EPT - EDG // NS)),
                       constant_values=fill)
    edg = jnp.stack([
        jnp.stack([_split(edge_index_pe[0], 0), _split(edge_index_pe[1], N)]),
        jnp.stack([_split(edge_index_ep[0], 0), _split(edge_index_ep[1], N)]),
    ]).reshape(NC * 2 * NS * NSUB, SUB)
    xtab = jnp.concatenate([
        yp.reshape(N, H, QW).transpose(1, 0, 2).reshape(H * N, QW),
        ye.reshape(N, H, QW).transpose(1, 0, 2).reshape(H * N, QW),
    ], axis=0)
    han, den = _sc_edge(atab, xtab, edg)
    erep = jnp.repeat(jnp.eye(16, dtype=jnp.float32), 64, axis=1)[:, :F]
    fin = _fin(han, den, erep, lin1_W, lin1_b.reshape(1, F))
    out_e = fin[:N]
    out_p = fin[NPAD:NPAD + N]
    return out_p, out_e
